# baseline (speedup 1.0000x reference)
import jax
import jax.numpy as jnp
from jax import lax
from jax.experimental import pallas as pl
from jax.experimental.pallas import tpu as pltpu
from jax.experimental.pallas import tpu_sc as plsc

_N = 10000
_E = 160000
_H = 32
_G = 256
_D = 128
_DE = 16
_HH = _H * _H

_GRP = 128
_NW = 32
_EPAD = 163840
_GPW = 40
_BE = 1280
_NEB = _E // _BE
_NEBP = _EPAD // _BE
_TSTRIPE = _N // 16

_NC, _NS = 2, 16


def _enc_body(x_ref, w_ref, b_ref, o_ref):
    o_ref[...] = jnp.dot(x_ref[...], w_ref[...],
                         preferred_element_type=jnp.float32) + b_ref[...]


def _edge_a_body(ef_ref, w1_ref, b1_ref, w2_ref, b2_ref, o_ref):
    t = jnp.dot(ef_ref[...], w1_ref[...], preferred_element_type=jnp.float32)
    t = jnp.maximum(t + b1_ref[...], 0.0)
    a = jnp.dot(t.astype(jnp.bfloat16), w2_ref[...],
                preferred_element_type=jnp.float32) + b2_ref[...]
    o_ref[...] = a.astype(jnp.bfloat16)


def _msg_body(a_ref, u_ref, o_ref):
    pid = pl.program_id(0)

    @pl.when(pid < _NEB)
    def _():
        u = u_ref[...].astype(jnp.bfloat16)

        def prod(g):
            u_rep = jnp.concatenate(
                [jnp.broadcast_to(u[:, 4 * g + t:4 * g + t + 1], (_BE, _H))
                 for t in range(4)], axis=1)
            return a_ref[:, g * 128:(g + 1) * 128] * u_rep

        acc = None
        for g in range(0, _HH // 128, 2):
            p = (prod(g) + prod(g + 1)).astype(jnp.float32)
            acc = p if acc is None else acc + p
        o_ref[...] = (acc[:, 0:_H] + acc[:, _H:2 * _H]
                      + acc[:, 2 * _H:3 * _H] + acc[:, 3 * _H:])

    @pl.when(pid >= _NEB)
    def _():
        o_ref[...] = jnp.zeros_like(o_ref)


def _gru_body(m0_ref, m1_ref, h_ref, wih_ref, whh_ref, bih_ref, bhh_ref,
              o_ref):
    m = m0_ref[...] + m1_ref[...]
    h = h_ref[...]
    gi = jnp.dot(m, wih_ref[...],
                 preferred_element_type=jnp.float32) + bih_ref[...]
    gh = jnp.dot(h, whh_ref[...],
                 preferred_element_type=jnp.float32) + bhh_ref[...]
    r = jax.nn.sigmoid(gi[:, :_H] + gh[:, :_H])
    z = jax.nn.sigmoid(gi[:, _H:2 * _H] + gh[:, _H:2 * _H])
    n = jnp.tanh(gi[:, 2 * _H:] + r * gh[:, 2 * _H:])
    o_ref[...] = (1.0 - z) * n + z * h


def _s2s_body(h_ref, bi_ref, lwih_ref, lwhh_ref, lbih_ref, lbhh_ref,
              w3_ref, b3_ref, w4_ref, b4_ref, o_ref):
    h = h_ref[...]
    bi = bi_ref[...]
    gid = lax.broadcasted_iota(jnp.int32, (_N, _G), 1)
    onehot = (bi == gid).astype(jnp.float32)
    s2s_h = jnp.zeros((_G, _H), jnp.float32)
    s2s_c = jnp.zeros((_G, _H), jnp.float32)
    r_out = jnp.zeros((_G, _H), jnp.float32)
    dn = (((0,), (0,)), ((), ()))
    for _ in range(4):
        q = jnp.dot(onehot, s2s_h, preferred_element_type=jnp.float32)
        e = jnp.sum(h * q, axis=1, keepdims=True)
        masked = jnp.where(bi == gid, e, -jnp.inf)
        emax = jnp.max(masked, axis=0, keepdims=True)
        emax = jnp.where(emax > -3e38, emax, 0.0)
        emax_n = jnp.sum(onehot * emax, axis=1, keepdims=True)
        ex = jnp.exp(e - emax_n)
        denom = lax.dot_general(ex, onehot, dimension_numbers=dn,
                                preferred_element_type=jnp.float32)
        den_n = jnp.sum(onehot * denom, axis=1, keepdims=True)
        att = ex / den_n
        r_out = lax.dot_general(onehot, att * h, dimension_numbers=dn,
                                preferred_element_type=jnp.float32)
        li = jnp.concatenate([s2s_h, r_out], axis=1)
        gates = (jnp.dot(li, lwih_ref[...], preferred_element_type=jnp.float32)
                 + lbih_ref[...]
                 + jnp.dot(s2s_h, lwhh_ref[...],
                           preferred_element_type=jnp.float32)
                 + lbhh_ref[...])
        ii = jax.nn.sigmoid(gates[:, :_H])
        ff = jax.nn.sigmoid(gates[:, _H:2 * _H])
        gg = jnp.tanh(gates[:, 2 * _H:3 * _H])
        oo = jax.nn.sigmoid(gates[:, 3 * _H:])
        s2s_c = ff * s2s_c + ii * gg
        s2s_h = oo * jnp.tanh(s2s_c)
    ge = jnp.concatenate([s2s_h, r_out], axis=1)
    t = jax.nn.relu(jnp.dot(ge, w3_ref[...],
                            preferred_element_type=jnp.float32) + b3_ref[...])
    o_ref[...] = jnp.dot(t, w4_ref[...],
                         preferred_element_type=jnp.float32) + b4_ref[...]


_K = _GPW // 2
_HROWS = _K * _GRP


def _sc_gather_body(h_hbm, src_hbm, out_hbm, idx_v, buf_v, sem):
    c = lax.axis_index("c")
    s = lax.axis_index("s")
    wid = s * _NC + c
    base = wid * _GPW * _GRP
    pltpu.sync_copy(src_hbm.at[pl.ds(wid * _GPW, _GPW)], idx_v)
    for half in range(2):
        handles = [
            pltpu.async_copy(h_hbm.at[idx_v.at[half * _K + t]],
                             buf_v.at[pl.ds(t * _GRP, _GRP)], sem)
            for t in range(_K)
        ]
        for hd in handles:
            hd.wait()
        pltpu.sync_copy(buf_v,
                        out_hbm.at[pl.ds(base + half * _HROWS, _HROWS)])


def _sc_scatter_body(msg_hbm, dst_hbm, zero_hbm, out_hbm, idx_v, buf_v, m_sh,
                     sem):
    c = lax.axis_index("c")
    s = lax.axis_index("s")
    wid = s * _NC + c
    base = wid * _GPW * _GRP

    @pl.when(s == 0)
    def _():
        pltpu.sync_copy(zero_hbm, m_sh)

    plsc.subcore_barrier()
    pltpu.sync_copy(dst_hbm.at[pl.ds(wid * _GPW, _GPW)], idx_v)
    for half in range(2):
        pltpu.sync_copy(
            msg_hbm.at[pl.ds(base + half * _HROWS, _HROWS)], buf_v)
        handles = [
            pltpu.async_copy(buf_v.at[pl.ds(t * _GRP, _GRP)],
                             m_sh.at[idx_v.at[half * _K + t]], sem, add=True)
            for t in range(_K)
        ]
        for hd in handles:
            hd.wait()
    plsc.subcore_barrier()
    pltpu.sync_copy(m_sh.at[pl.ds(s * _TSTRIPE, _TSTRIPE)],
                    out_hbm.at[c, pl.ds(s * _TSTRIPE, _TSTRIPE)])


def _sc_mesh():
    return plsc.VectorSubcoreMesh(core_axis_name="c", subcore_axis_name="s",
                                  num_cores=_NC, num_subcores=_NS)


def kernel(node_features, edge_index, edge_features, batch_indices,
           W_enc, b_enc, W1, b1, W2, b2, gw_ih, gw_hh, gb_ih, gb_hh,
           lw_ih, lw_hh, lb_ih, lb_hh, W3, b3, W4, b4):
    f32 = jnp.float32
    src = edge_index[0]
    dst = edge_index[1]
    pad = jnp.zeros((_EPAD - _E,), jnp.int32)
    src_r = jnp.concatenate([src, pad]).reshape(_EPAD // _GRP, _GRP)
    dst_r = jnp.concatenate([dst, pad]).reshape(_EPAD // _GRP, _GRP)
    bi = batch_indices.reshape(_N, 1)

    W2p = W2.reshape(64, _H, _H).transpose(0, 2, 1).reshape(64, _HH)
    b2p = b2.reshape(_H, _H).T.reshape(1, _HH)
    W4p = jnp.zeros((_H, 128), f32).at[:, :3].set(W4)
    b4p = jnp.zeros((1, 128), f32).at[:, :3].set(b4)
    zero_n = jnp.zeros((_N, _H), f32)

    h = pl.pallas_call(
        _enc_body,
        out_shape=jax.ShapeDtypeStruct((_N, _H), f32),
    )(node_features, W_enc, b_enc.reshape(1, _H))

    a_mat = pl.pallas_call(
        _edge_a_body,
        grid=(_NEB,),
        in_specs=[
            pl.BlockSpec((_BE, _DE), lambda i: (i, 0)),
            pl.BlockSpec((_DE, 64), lambda i: (0, 0)),
            pl.BlockSpec((1, 64), lambda i: (0, 0)),
            pl.BlockSpec((64, _HH), lambda i: (0, 0)),
            pl.BlockSpec((1, _HH), lambda i: (0, 0)),
        ],
        out_specs=pl.BlockSpec((_BE, _HH), lambda i: (i, 0)),
        out_shape=jax.ShapeDtypeStruct((_EPAD, _HH), jnp.bfloat16),
    )(edge_features, W1, b1.reshape(1, 64), W2p.astype(jnp.bfloat16), b2p)

    gather_fn = pl.kernel(
        _sc_gather_body,
        out_type=jax.ShapeDtypeStruct((_EPAD, _H), f32),
        mesh=_sc_mesh(),
        compiler_params=pltpu.CompilerParams(use_tc_tiling_on_sc=False),
        scratch_types=[
            pltpu.VMEM((_GPW, _GRP), jnp.int32),
            pltpu.VMEM((_HROWS, _H), f32),
            pltpu.SemaphoreType.DMA,
        ],
    )

    scatter_fn = pl.kernel(
        _sc_scatter_body,
        out_type=jax.ShapeDtypeStruct((_NC, _N, _H), f32),
        mesh=_sc_mesh(),
        compiler_params=pltpu.CompilerParams(use_tc_tiling_on_sc=False),
        scratch_types=[
            pltpu.VMEM((_GPW, _GRP), jnp.int32),
            pltpu.VMEM((_HROWS, _H), f32),
            pltpu.VMEM_SHARED((_N, _H), f32),
            pltpu.SemaphoreType.DMA,
        ],
    )

    msg_fn = pl.pallas_call(
        _msg_body,
        grid=(_NEBP,),
        in_specs=[
            pl.BlockSpec((_BE, _HH), lambda i: (i, 0)),
            pl.BlockSpec((_BE, _H), lambda i: (i, 0)),
        ],
        out_specs=pl.BlockSpec((_BE, _H), lambda i: (i, 0)),
        out_shape=jax.ShapeDtypeStruct((_EPAD, _H), f32),
    )

    gru_fn = pl.pallas_call(
        _gru_body,
        out_shape=jax.ShapeDtypeStruct((_N, _H), f32),
    )
    gwih_t = gw_ih.T
    gwhh_t = gw_hh.T
    gbih = gb_ih.reshape(1, 3 * _H)
    gbhh = gb_hh.reshape(1, 3 * _H)

    for _ in range(3):
        u = gather_fn(h, src_r)
        msg = msg_fn(a_mat, u)
        m2 = scatter_fn(msg, dst_r, zero_n)
        h = gru_fn(m2[0], m2[1], h, gwih_t, gwhh_t, gbih, gbhh)

    out = pl.pallas_call(
        _s2s_body,
        out_shape=jax.ShapeDtypeStruct((_G, 128), f32),
    )(h, bi, lw_ih.T, lw_hh.T, lb_ih.reshape(1, 4 * _H),
      lb_hh.reshape(1, 4 * _H), W3, b3.reshape(1, _H), W4p, b4p)
    return out[:, :3]


# speedup vs baseline: 1.0120x; 1.0120x over previous
import jax
import jax.numpy as jnp
from jax import lax
from jax.experimental import pallas as pl
from jax.experimental.pallas import tpu as pltpu
from jax.experimental.pallas import tpu_sc as plsc

_N = 10000
_E = 160000
_H = 32
_G = 256
_D = 128
_DE = 16
_HH = _H * _H

_GRP = 128
_NW = 32
_EPAD = 163840
_GPW = 40
_BE = 1280
_NEB = _E // _BE
_NEBP = _EPAD // _BE
_TSTRIPE = _N // 16

_NC, _NS = 2, 16


def _enc_body(x_ref, w_ref, b_ref, o_ref):
    o_ref[...] = jnp.dot(x_ref[...], w_ref[...],
                         preferred_element_type=jnp.float32) + b_ref[...]


def _edge_a_body(ef_ref, w1_ref, b1_ref, w2_ref, b2_ref, o_ref):
    t = jnp.dot(ef_ref[...], w1_ref[...], preferred_element_type=jnp.float32)
    t = jnp.maximum(t + b1_ref[...], 0.0)
    a = jnp.dot(t.astype(jnp.bfloat16), w2_ref[...],
                preferred_element_type=jnp.float32) + b2_ref[...]
    o_ref[...] = a.astype(jnp.bfloat16)


def _msg_body(a_ref, u_ref, o_ref):
    pid = pl.program_id(0)

    @pl.when(pid < _NEB)
    def _():
        u = u_ref[...].astype(jnp.bfloat16)
        acc = None
        for g in range(_HH // 128):
            u_rep = jnp.concatenate(
                [jnp.broadcast_to(u[:, 4 * g + t:4 * g + t + 1], (_BE, _H))
                 for t in range(4)], axis=1)
            p = (a_ref[:, g * 128:(g + 1) * 128] * u_rep).astype(jnp.float32)
            acc = p if acc is None else acc + p
        o_ref[...] = (acc[:, 0:_H] + acc[:, _H:2 * _H]
                      + acc[:, 2 * _H:3 * _H] + acc[:, 3 * _H:])

    @pl.when(pid >= _NEB)
    def _():
        o_ref[...] = jnp.zeros_like(o_ref)


def _gru_body(m0_ref, m1_ref, h_ref, wih_ref, whh_ref, bih_ref, bhh_ref,
              o_ref):
    m = m0_ref[...] + m1_ref[...]
    h = h_ref[...]
    gi = jnp.dot(m, wih_ref[...],
                 preferred_element_type=jnp.float32) + bih_ref[...]
    gh = jnp.dot(h, whh_ref[...],
                 preferred_element_type=jnp.float32) + bhh_ref[...]
    r = jax.nn.sigmoid(gi[:, :_H] + gh[:, :_H])
    z = jax.nn.sigmoid(gi[:, _H:2 * _H] + gh[:, _H:2 * _H])
    n = jnp.tanh(gi[:, 2 * _H:] + r * gh[:, 2 * _H:])
    o_ref[...] = (1.0 - z) * n + z * h


def _s2s_body(h_ref, bi_ref, lwih_ref, lwhh_ref, lbih_ref, lbhh_ref,
              w3_ref, b3_ref, w4_ref, b4_ref, o_ref):
    h = h_ref[...]
    bi = bi_ref[...]
    gid = lax.broadcasted_iota(jnp.int32, (_N, _G), 1)
    onehot = (bi == gid).astype(jnp.float32)
    s2s_h = jnp.zeros((_G, _H), jnp.float32)
    s2s_c = jnp.zeros((_G, _H), jnp.float32)
    r_out = jnp.zeros((_G, _H), jnp.float32)
    dn = (((0,), (0,)), ((), ()))
    for _ in range(4):
        q = jnp.dot(onehot, s2s_h, preferred_element_type=jnp.float32)
        e = jnp.sum(h * q, axis=1, keepdims=True)
        masked = jnp.where(bi == gid, e, -jnp.inf)
        emax = jnp.max(masked, axis=0, keepdims=True)
        emax = jnp.where(emax > -3e38, emax, 0.0)
        emax_n = jnp.sum(onehot * emax, axis=1, keepdims=True)
        ex = jnp.exp(e - emax_n)
        denom = lax.dot_general(ex, onehot, dimension_numbers=dn,
                                preferred_element_type=jnp.float32)
        den_n = jnp.sum(onehot * denom, axis=1, keepdims=True)
        att = ex / den_n
        r_out = lax.dot_general(onehot, att * h, dimension_numbers=dn,
                                preferred_element_type=jnp.float32)
        li = jnp.concatenate([s2s_h, r_out], axis=1)
        gates = (jnp.dot(li, lwih_ref[...], preferred_element_type=jnp.float32)
                 + lbih_ref[...]
                 + jnp.dot(s2s_h, lwhh_ref[...],
                           preferred_element_type=jnp.float32)
                 + lbhh_ref[...])
        ii = jax.nn.sigmoid(gates[:, :_H])
        ff = jax.nn.sigmoid(gates[:, _H:2 * _H])
        gg = jnp.tanh(gates[:, 2 * _H:3 * _H])
        oo = jax.nn.sigmoid(gates[:, 3 * _H:])
        s2s_c = ff * s2s_c + ii * gg
        s2s_h = oo * jnp.tanh(s2s_c)
    ge = jnp.concatenate([s2s_h, r_out], axis=1)
    t = jax.nn.relu(jnp.dot(ge, w3_ref[...],
                            preferred_element_type=jnp.float32) + b3_ref[...])
    o_ref[...] = jnp.dot(t, w4_ref[...],
                         preferred_element_type=jnp.float32) + b4_ref[...]


_K = _GPW // 2
_HROWS = _K * _GRP


def _sc_gather_body(h_hbm, src_hbm, out_hbm, idx_v, buf_v, sem):
    c = lax.axis_index("c")
    s = lax.axis_index("s")
    wid = s * _NC + c
    base = wid * _GPW * _GRP
    pltpu.sync_copy(src_hbm.at[pl.ds(wid * _GPW, _GPW)], idx_v)
    for half in range(2):
        handles = [
            pltpu.async_copy(h_hbm.at[idx_v.at[half * _K + t]],
                             buf_v.at[pl.ds(t * _GRP, _GRP)], sem)
            for t in range(_K)
        ]
        for hd in handles:
            hd.wait()
        pltpu.sync_copy(buf_v,
                        out_hbm.at[pl.ds(base + half * _HROWS, _HROWS)])


def _sc_scatter_body(msg_hbm, dst_hbm, zero_hbm, out_hbm, idx_v, buf_v, m_sh,
                     sem):
    c = lax.axis_index("c")
    s = lax.axis_index("s")
    wid = s * _NC + c
    base = wid * _GPW * _GRP

    @pl.when(s == 0)
    def _():
        pltpu.sync_copy(zero_hbm, m_sh)

    plsc.subcore_barrier()
    pltpu.sync_copy(dst_hbm.at[pl.ds(wid * _GPW, _GPW)], idx_v)
    for half in range(2):
        pltpu.sync_copy(
            msg_hbm.at[pl.ds(base + half * _HROWS, _HROWS)], buf_v)
        handles = [
            pltpu.async_copy(buf_v.at[pl.ds(t * _GRP, _GRP)],
                             m_sh.at[idx_v.at[half * _K + t]], sem, add=True)
            for t in range(_K)
        ]
        for hd in handles:
            hd.wait()
    plsc.subcore_barrier()
    pltpu.sync_copy(m_sh.at[pl.ds(s * _TSTRIPE, _TSTRIPE)],
                    out_hbm.at[c, pl.ds(s * _TSTRIPE, _TSTRIPE)])


def _sc_mesh():
    return plsc.VectorSubcoreMesh(core_axis_name="c", subcore_axis_name="s",
                                  num_cores=_NC, num_subcores=_NS)


def kernel(node_features, edge_index, edge_features, batch_indices,
           W_enc, b_enc, W1, b1, W2, b2, gw_ih, gw_hh, gb_ih, gb_hh,
           lw_ih, lw_hh, lb_ih, lb_hh, W3, b3, W4, b4):
    f32 = jnp.float32
    src = edge_index[0]
    dst = edge_index[1]
    pad = jnp.zeros((_EPAD - _E,), jnp.int32)
    src_r = jnp.concatenate([src, pad]).reshape(_EPAD // _GRP, _GRP)
    dst_r = jnp.concatenate([dst, pad]).reshape(_EPAD // _GRP, _GRP)
    bi = batch_indices.reshape(_N, 1)

    W2p = W2.reshape(64, _H, _H).transpose(0, 2, 1).reshape(64, _HH)
    b2p = b2.reshape(_H, _H).T.reshape(1, _HH)
    W4p = jnp.zeros((_H, 128), f32).at[:, :3].set(W4)
    b4p = jnp.zeros((1, 128), f32).at[:, :3].set(b4)
    zero_n = jnp.zeros((_N, _H), f32)

    h = pl.pallas_call(
        _enc_body,
        out_shape=jax.ShapeDtypeStruct((_N, _H), f32),
    )(node_features, W_enc, b_enc.reshape(1, _H))

    a_mat = pl.pallas_call(
        _edge_a_body,
        grid=(_NEB,),
        in_specs=[
            pl.BlockSpec((_BE, _DE), lambda i: (i, 0)),
            pl.BlockSpec((_DE, 64), lambda i: (0, 0)),
            pl.BlockSpec((1, 64), lambda i: (0, 0)),
            pl.BlockSpec((64, _HH), lambda i: (0, 0)),
            pl.BlockSpec((1, _HH), lambda i: (0, 0)),
        ],
        out_specs=pl.BlockSpec((_BE, _HH), lambda i: (i, 0)),
        out_shape=jax.ShapeDtypeStruct((_EPAD, _HH), jnp.bfloat16),
    )(edge_features, W1, b1.reshape(1, 64), W2p.astype(jnp.bfloat16), b2p)

    gather_fn = pl.kernel(
        _sc_gather_body,
        out_type=jax.ShapeDtypeStruct((_EPAD, _H), f32),
        mesh=_sc_mesh(),
        compiler_params=pltpu.CompilerParams(use_tc_tiling_on_sc=False),
        scratch_types=[
            pltpu.VMEM((_GPW, _GRP), jnp.int32),
            pltpu.VMEM((_HROWS, _H), f32),
            pltpu.SemaphoreType.DMA,
        ],
    )

    scatter_fn = pl.kernel(
        _sc_scatter_body,
        out_type=jax.ShapeDtypeStruct((_NC, _N, _H), f32),
        mesh=_sc_mesh(),
        compiler_params=pltpu.CompilerParams(use_tc_tiling_on_sc=False),
        scratch_types=[
            pltpu.VMEM((_GPW, _GRP), jnp.int32),
            pltpu.VMEM((_HROWS, _H), f32),
            pltpu.VMEM_SHARED((_N, _H), f32),
            pltpu.SemaphoreType.DMA,
        ],
    )

    msg_fn = pl.pallas_call(
        _msg_body,
        grid=(_NEBP,),
        in_specs=[
            pl.BlockSpec((_BE, _HH), lambda i: (i, 0)),
            pl.BlockSpec((_BE, _H), lambda i: (i, 0)),
        ],
        out_specs=pl.BlockSpec((_BE, _H), lambda i: (i, 0)),
        out_shape=jax.ShapeDtypeStruct((_EPAD, _H), f32),
    )

    gru_fn = pl.pallas_call(
        _gru_body,
        out_shape=jax.ShapeDtypeStruct((_N, _H), f32),
    )
    gwih_t = gw_ih.T
    gwhh_t = gw_hh.T
    gbih = gb_ih.reshape(1, 3 * _H)
    gbhh = gb_hh.reshape(1, 3 * _H)

    for _ in range(3):
        u = gather_fn(h, src_r)
        msg = msg_fn(a_mat, u)
        m2 = scatter_fn(msg, dst_r, zero_n)
        h = gru_fn(m2[0], m2[1], h, gwih_t, gwhh_t, gbih, gbhh)

    out = pl.pallas_call(
        _s2s_body,
        out_shape=jax.ShapeDtypeStruct((_G, 128), f32),
    )(h, bi, lw_ih.T, lw_hh.T, lb_ih.reshape(1, 4 * _H),
      lb_hh.reshape(1, 4 * _H), W3, b3.reshape(1, _H), W4p, b4p)
    return out[:, :3]
